# Initial kernel scaffold; baseline (speedup 1.0000x reference)
import jax
import jax.numpy as jnp
from jax.experimental import pallas as pl


def kernel(encoded, masks):
    raise NotImplementedError("write your pallas kernel here")


# speedup vs baseline: 4.2042x; 4.2042x over previous
import functools

import jax
import jax.numpy as jnp
from jax import lax
from jax.experimental import pallas as pl
from jax.experimental.pallas import tpu as pltpu
from jax.experimental.pallas import tpu_sc as plsc

B, C, H, W = 4, 96, 384, 384
HW = H * W
V = 32
NSEG = V + 1
NC, NS, L = 2, 16, 16
NW = NC * NS
PPT = HW // NW
P = 256
NCHUNK = PPT // P
GRP = P // L
ACCW = C * NSEG * L
PART = NSEG * C
CB = C // L


def _segmax_body(enc, msk, part_out, dbuf, mbuf, acc, pbuf):
    wid = lax.axis_index("s") * NC + lax.axis_index("c")
    lane = lax.broadcasted_iota(jnp.int32, (L,), 0)
    ninf = jnp.full((L,), -jnp.inf, dtype=jnp.float32)
    lane528 = lane * (NSEG * L)

    def batch_body(b, carry):
        @plsc.parallel_loop(0, ACCW // L, 1, unroll=8)
        def _init(i):
            acc[pl.ds(i * L, L)] = ninf

        base = wid * PPT

        def chunk_body(k, carry2):
            off = base + k * P
            pltpu.sync_copy(enc.at[b, :, pl.ds(off, P)], dbuf)
            pltpu.sync_copy(msk.at[b, pl.ds(off, P)], mbuf)

            def grp_body(g, carry3):
                m = mbuf[pl.ds(g * L, L)]
                idx0 = m * L + lane
                pvec = jnp.full((L,), g * L, jnp.int32) + lane

                @plsc.parallel_loop(0, C, 1, unroll=4)
                def _cbody(c):
                    d = plsc.load_gather(dbuf, [jnp.full((L,), c, jnp.int32), pvec])
                    idx = idx0 + jnp.full((L,), c * (NSEG * L), jnp.int32)
                    old = plsc.load_gather(acc, [idx])
                    plsc.store_scatter(acc, [idx], jnp.maximum(old, d))

                return carry3

            lax.fori_loop(0, GRP, grp_body, 0)
            return carry2

        lax.fori_loop(0, NCHUNK, chunk_body, 0)

        @plsc.parallel_loop(0, NSEG * CB, 1, unroll=2)
        def _flush(t):
            v = t // CB
            cb = t % CB
            sbase = cb * (L * NSEG * L) + v * L
            red = plsc.load_gather(acc, [lane528 + jnp.full((L,), sbase, jnp.int32)])
            for j in range(1, L):
                red = jnp.maximum(
                    red,
                    plsc.load_gather(acc, [lane528 + jnp.full((L,), sbase + j, jnp.int32)]),
                )
            pbuf[pl.ds(v * C + cb * L, L)] = red

        pltpu.sync_copy(pbuf, part_out.at[pl.ds((b * NW + wid) * PART, PART)])
        return carry

    lax.fori_loop(0, B, batch_body, 0)


def _merge_body(part4, out3, sbuf, obuf):
    wid = lax.axis_index("s") * NC + lax.axis_index("c")
    lane = lax.broadcasted_iota(jnp.int32, (L,), 0)

    @pl.when(wid < B * CB)
    def _():
        b = wid // CB
        cb = wid % CB
        pltpu.sync_copy(part4.at[b, :, :, pl.ds(cb * L, L)], sbuf)
        for v in range(1, NSEG):
            red = sbuf[0, v, :]
            for t in range(1, NW):
                red = jnp.maximum(red, sbuf[t, v, :])
            plsc.store_scatter(obuf, [lane, jnp.full((L,), v - 1, jnp.int32)], red)
        pltpu.sync_copy(obuf, out3.at[b, pl.ds(cb * L, L), :])


def kernel(encoded, masks):
    enc = encoded.reshape(B, C, HW)
    msk = masks.reshape(B, HW)
    mesh = plsc.VectorSubcoreMesh(core_axis_name="c", subcore_axis_name="s")

    params = pltpu.CompilerParams(
        use_tc_tiling_on_sc=False, needs_layout_passes=False
    )

    seg = pl.kernel(
        _segmax_body,
        out_type=jax.ShapeDtypeStruct((B * NW * PART,), jnp.float32),
        mesh=mesh,
        compiler_params=params,
        scratch_types=[
            pltpu.VMEM((C, P), jnp.float32),
            pltpu.VMEM((P,), jnp.int32),
            pltpu.VMEM((ACCW,), jnp.float32),
            pltpu.VMEM((PART,), jnp.float32),
        ],
    )
    partials = seg(enc, msk).reshape(B, NW, NSEG, C)

    merge = pl.kernel(
        _merge_body,
        out_type=jax.ShapeDtypeStruct((B, C, V), jnp.float32),
        mesh=mesh,
        compiler_params=params,
        scratch_types=[
            pltpu.VMEM((NW, NSEG, L), jnp.float32),
            pltpu.VMEM((L, V), jnp.float32),
        ],
    )
    out3 = merge(partials)
    return out3[..., None]


# speedup vs baseline: 6.6534x; 1.5826x over previous
import functools

import jax
import jax.numpy as jnp
from jax import lax
from jax.experimental import pallas as pl
from jax.experimental.pallas import tpu as pltpu
from jax.experimental.pallas import tpu_sc as plsc

B, C, H, W = 4, 96, 384, 384
HW = H * W
V = 32
NSEG = V + 1
NC, NS, L = 2, 16, 16
NW = NC * NS
PPT = HW // NW
P = 256
NCHUNK = PPT // P
GRP = P // L
ACCW = C * NSEG * L
PART = NSEG * C
CB = C // L


def _segmax_body(enc, msk, part_out, dbuf, mbuf, acc, pbuf):
    wid = lax.axis_index("s") * NC + lax.axis_index("c")
    lane = lax.broadcasted_iota(jnp.int32, (L,), 0)
    ninf = jnp.full((L,), -jnp.inf, dtype=jnp.float32)
    lane528 = lane * (NSEG * L)

    def batch_body(b, carry):
        @plsc.parallel_loop(0, ACCW // L, 1, unroll=8)
        def _init(i):
            acc[pl.ds(i * L, L)] = ninf

        base = wid * PPT

        def chunk_body(k, carry2):
            off = base + k * P
            pltpu.sync_copy(enc.at[b, :, pl.ds(off, P)], dbuf)
            pltpu.sync_copy(msk.at[b, pl.ds(off, P)], mbuf)

            def grp_body(g, carry3):
                m = mbuf[pl.ds(g * L, L)]
                idx0 = m * L + lane
                pvec = jnp.full((L,), g * L, jnp.int32) + lane

                @plsc.parallel_loop(0, C, 1, unroll=4)
                def _cbody(c):
                    d = plsc.load_gather(dbuf, [jnp.full((L,), c, jnp.int32), pvec])
                    idx = idx0 + jnp.full((L,), c * (NSEG * L), jnp.int32)
                    old = plsc.load_gather(acc, [idx])
                    plsc.store_scatter(acc, [idx], jnp.maximum(old, d))

                return carry3

            lax.fori_loop(0, GRP, grp_body, 0)
            return carry2

        lax.fori_loop(0, NCHUNK, chunk_body, 0)

        @plsc.parallel_loop(0, NSEG * CB, 1, unroll=2)
        def _flush(t):
            v = t // CB
            cb = t % CB
            sbase = cb * (L * NSEG * L) + v * L
            red = plsc.load_gather(acc, [lane528 + jnp.full((L,), sbase, jnp.int32)])
            for j in range(1, L):
                red = jnp.maximum(
                    red,
                    plsc.load_gather(acc, [lane528 + jnp.full((L,), sbase + j, jnp.int32)]),
                )
            pbuf[pl.ds(v * C + cb * L, L)] = red

        pltpu.sync_copy(pbuf, part_out.at[pl.ds((b * NW + wid) * PART, PART)])
        return carry

    lax.fori_loop(0, B, batch_body, 0)


def _merge_body(part4, out3, sbuf, obuf):
    wid = lax.axis_index("s") * NC + lax.axis_index("c")
    lane = lax.broadcasted_iota(jnp.int32, (L,), 0)

    @pl.when(wid < B * CB)
    def _():
        b = wid // CB
        cb = wid % CB
        pltpu.sync_copy(part4.at[b, :, :, pl.ds(cb * L, L)], sbuf)
        for v in range(1, NSEG):
            red = sbuf[0, v, :]
            for t in range(1, NW):
                red = jnp.maximum(red, sbuf[t, v, :])
            plsc.store_scatter(obuf, [lane, jnp.full((L,), v - 1, jnp.int32)], red)
        pltpu.sync_copy(obuf, out3.at[b, pl.ds(cb * L, L), :])


def kernel(encoded, masks):
    TH, TW = H // 8, W // 128
    enc = (
        encoded.reshape(B, C, TH, 8, TW, 128)
        .transpose(0, 1, 2, 4, 3, 5)
        .reshape(B, C, HW)
    )
    msk = (
        masks.reshape(B, TH, 8, TW, 128)
        .transpose(0, 1, 3, 2, 4)
        .reshape(B, HW)
    )
    mesh = plsc.VectorSubcoreMesh(core_axis_name="c", subcore_axis_name="s")

    params = pltpu.CompilerParams(
        use_tc_tiling_on_sc=False, needs_layout_passes=False
    )

    seg = pl.kernel(
        _segmax_body,
        out_type=jax.ShapeDtypeStruct((B * NW * PART,), jnp.float32),
        mesh=mesh,
        compiler_params=params,
        scratch_types=[
            pltpu.VMEM((C, P), jnp.float32),
            pltpu.VMEM((P,), jnp.int32),
            pltpu.VMEM((ACCW,), jnp.float32),
            pltpu.VMEM((PART,), jnp.float32),
        ],
    )
    partials = seg(enc, msk).reshape(B, NW, NSEG, C)

    merge = pl.kernel(
        _merge_body,
        out_type=jax.ShapeDtypeStruct((B, C, V), jnp.float32),
        mesh=mesh,
        compiler_params=params,
        scratch_types=[
            pltpu.VMEM((NW, NSEG, L), jnp.float32),
            pltpu.VMEM((L, V), jnp.float32),
        ],
    )
    out3 = merge(partials)
    return out3[..., None]


# speedup vs baseline: 11.0947x; 1.6675x over previous
import functools

import jax
import jax.numpy as jnp
from jax import lax
from jax.experimental import pallas as pl
from jax.experimental.pallas import tpu as pltpu
from jax.experimental.pallas import tpu_sc as plsc

B, C, H, W = 4, 96, 384, 384
HW = H * W
V = 32
NSEG = V + 1
NC, NS, L = 2, 16, 16
NW = NC * NS
PPT = HW // NW
P = 256
NCHUNK = PPT // P
GRP = P // L
ACCW = C * NSEG * L
PART = NSEG * C
CB = C // L


def _segmax_body(
    enc, msk, part_out,
    dbuf0, dbuf1, mbuf0, mbuf1, acc, pbuf,
    semd0, semd1, semm0, semm1,
):
    wid = lax.axis_index("s") * NC + lax.axis_index("c")
    lane = lax.broadcasted_iota(jnp.int32, (L,), 0)
    ninf = jnp.full((L,), -jnp.inf, dtype=jnp.float32)
    lane528 = lane * (NSEG * L)

    def start(b, k, dref, mref, semd, semm):
        off = wid * PPT + k * P
        pltpu.async_copy(enc.at[b, :, pl.ds(off, P)], dref, semd)
        pltpu.async_copy(msk.at[b, pl.ds(off, P)], mref, semm)

    def wait(dref, mref, semd, semm):
        pltpu.make_async_copy(enc.at[0, :, pl.ds(0, P)], dref, semd).wait()
        pltpu.make_async_copy(msk.at[0, pl.ds(0, P)], mref, semm).wait()

    def process(dref, mref):
        def grp_body(g, carry3):
            m = mref[pl.ds(g * L, L)]
            idx0 = m * L + lane
            pvec = jnp.full((L,), g * L, jnp.int32) + lane

            @plsc.parallel_loop(0, C, 1, unroll=4)
            def _cbody(c):
                d = plsc.load_gather(dref, [jnp.full((L,), c, jnp.int32), pvec])
                idx = idx0 + jnp.full((L,), c * (NSEG * L), jnp.int32)
                old = plsc.load_gather(acc, [idx])
                plsc.store_scatter(acc, [idx], jnp.maximum(old, d))

            return carry3

        lax.fori_loop(0, GRP, grp_body, 0)

    def batch_body(b, carry):
        @plsc.parallel_loop(0, ACCW // L, 1, unroll=8)
        def _init(i):
            acc[pl.ds(i * L, L)] = ninf

        start(b, 0, dbuf0, mbuf0, semd0, semm0)

        def pair_body(kk, carry2):
            wait(dbuf0, mbuf0, semd0, semm0)
            start(b, 2 * kk + 1, dbuf1, mbuf1, semd1, semm1)
            process(dbuf0, mbuf0)
            wait(dbuf1, mbuf1, semd1, semm1)

            @pl.when(kk < NCHUNK // 2 - 1)
            def _():
                start(b, 2 * kk + 2, dbuf0, mbuf0, semd0, semm0)

            process(dbuf1, mbuf1)
            return carry2

        lax.fori_loop(0, NCHUNK // 2, pair_body, 0)

        @plsc.parallel_loop(0, NSEG * CB, 1, unroll=2)
        def _flush(t):
            v = t // CB
            cb = t % CB
            sbase = cb * (L * NSEG * L) + v * L
            red = plsc.load_gather(acc, [lane528 + jnp.full((L,), sbase, jnp.int32)])
            for j in range(1, L):
                red = jnp.maximum(
                    red,
                    plsc.load_gather(acc, [lane528 + jnp.full((L,), sbase + j, jnp.int32)]),
                )
            pbuf[pl.ds(v * C + cb * L, L)] = red

        pltpu.sync_copy(pbuf, part_out.at[pl.ds((b * NW + wid) * PART, PART)])
        return carry

    lax.fori_loop(0, B, batch_body, 0)


def _merge_body(part4, out3, sbuf, obuf):
    wid = lax.axis_index("s") * NC + lax.axis_index("c")
    lane = lax.broadcasted_iota(jnp.int32, (L,), 0)

    @pl.when(wid < B * CB)
    def _():
        b = wid // CB
        cb = wid % CB
        pltpu.sync_copy(part4.at[b, :, :, pl.ds(cb * L, L)], sbuf)
        for v in range(1, NSEG):
            red = sbuf[0, v, :]
            for t in range(1, NW):
                red = jnp.maximum(red, sbuf[t, v, :])
            plsc.store_scatter(obuf, [lane, jnp.full((L,), v - 1, jnp.int32)], red)
        pltpu.sync_copy(obuf, out3.at[b, pl.ds(cb * L, L), :])


def kernel(encoded, masks):
    TH, TW = H // 8, W // 128
    enc = (
        encoded.reshape(B, C, TH, 8, TW, 128)
        .transpose(0, 1, 2, 4, 3, 5)
        .reshape(B, C, HW)
    )
    msk = (
        masks.reshape(B, TH, 8, TW, 128)
        .transpose(0, 1, 3, 2, 4)
        .reshape(B, HW)
    )
    mesh = plsc.VectorSubcoreMesh(core_axis_name="c", subcore_axis_name="s")

    params = pltpu.CompilerParams(
        use_tc_tiling_on_sc=False, needs_layout_passes=False
    )

    seg = pl.kernel(
        _segmax_body,
        out_type=jax.ShapeDtypeStruct((B * NW * PART,), jnp.float32),
        mesh=mesh,
        compiler_params=params,
        scratch_types=[
            pltpu.VMEM((C, P), jnp.float32),
            pltpu.VMEM((C, P), jnp.float32),
            pltpu.VMEM((P,), jnp.int32),
            pltpu.VMEM((P,), jnp.int32),
            pltpu.VMEM((ACCW,), jnp.float32),
            pltpu.VMEM((PART,), jnp.float32),
            pltpu.SemaphoreType.DMA,
            pltpu.SemaphoreType.DMA,
            pltpu.SemaphoreType.DMA,
            pltpu.SemaphoreType.DMA,
        ],
    )
    partials = seg(enc, msk).reshape(B, NW, NSEG, C)

    merge = pl.kernel(
        _merge_body,
        out_type=jax.ShapeDtypeStruct((B, C, V), jnp.float32),
        mesh=mesh,
        compiler_params=params,
        scratch_types=[
            pltpu.VMEM((NW, NSEG, L), jnp.float32),
            pltpu.VMEM((L, V), jnp.float32),
        ],
    )
    out3 = merge(partials)
    return out3[..., None]


# speedup vs baseline: 11.9185x; 1.0743x over previous
import functools

import jax
import jax.numpy as jnp
from jax import lax
from jax.experimental import pallas as pl
from jax.experimental.pallas import tpu as pltpu
from jax.experimental.pallas import tpu_sc as plsc

B, C, H, W = 4, 96, 384, 384
HW = H * W
V = 32
NSEG = V + 1
NC, NS, L = 2, 16, 16
NW = NC * NS
PPT = HW // NW
P = 256
NCHUNK = PPT // P
GRP = P // L
ACCW = C * NSEG * L
PART = NSEG * C
CB = C // L


def _segmax_body(
    enc, msk, part_out,
    dbuf0, dbuf1, mbuf0, mbuf1, acc, pbuf,
    semd0, semd1, semm0, semm1,
):
    wid = lax.axis_index("s") * NC + lax.axis_index("c")
    lane = lax.broadcasted_iota(jnp.int32, (L,), 0)
    ninf = jnp.full((L,), -jnp.inf, dtype=jnp.float32)
    lane528 = lane * (NSEG * L)

    def start(b, k, dref, mref, semd, semm):
        off = wid * PPT + k * P
        pltpu.async_copy(enc.at[b, :, pl.ds(off, P)], dref, semd)
        pltpu.async_copy(msk.at[b, pl.ds(off, P)], mref, semm)

    def wait(dref, mref, semd, semm):
        pltpu.make_async_copy(enc.at[0, :, pl.ds(0, P)], dref, semd).wait()
        pltpu.make_async_copy(msk.at[0, pl.ds(0, P)], mref, semm).wait()

    def process(dref, mref):
        def grp_body(g, carry3):
            m = mref[pl.ds(g * L, L)]
            idx0 = m * L + lane

            @plsc.parallel_loop(0, C, 1, unroll=4)
            def _cbody(c):
                d = dref[c, pl.ds(g * L, L)]
                accv = acc.at[pl.ds(c * (NSEG * L), NSEG * L)]
                old = plsc.load_gather(accv, [idx0])
                plsc.store_scatter(accv, [idx0], jnp.maximum(old, d))

            return carry3

        lax.fori_loop(0, GRP, grp_body, 0)

    def batch_body(b, carry):
        @plsc.parallel_loop(0, ACCW // L, 1, unroll=8)
        def _init(i):
            acc[pl.ds(i * L, L)] = ninf

        start(b, 0, dbuf0, mbuf0, semd0, semm0)

        def pair_body(kk, carry2):
            wait(dbuf0, mbuf0, semd0, semm0)
            start(b, 2 * kk + 1, dbuf1, mbuf1, semd1, semm1)
            process(dbuf0, mbuf0)
            wait(dbuf1, mbuf1, semd1, semm1)

            @pl.when(kk < NCHUNK // 2 - 1)
            def _():
                start(b, 2 * kk + 2, dbuf0, mbuf0, semd0, semm0)

            process(dbuf1, mbuf1)
            return carry2

        lax.fori_loop(0, NCHUNK // 2, pair_body, 0)

        @plsc.parallel_loop(0, NSEG * CB, 1, unroll=2)
        def _flush(t):
            v = t // CB
            cb = t % CB
            sbase = cb * (L * NSEG * L) + v * L
            red = plsc.load_gather(acc, [lane528 + jnp.full((L,), sbase, jnp.int32)])
            for j in range(1, L):
                red = jnp.maximum(
                    red,
                    plsc.load_gather(acc, [lane528 + jnp.full((L,), sbase + j, jnp.int32)]),
                )
            pbuf[pl.ds(v * C + cb * L, L)] = red

        pltpu.sync_copy(pbuf, part_out.at[pl.ds((b * NW + wid) * PART, PART)])
        return carry

    lax.fori_loop(0, B, batch_body, 0)


def _merge_body(part4, out3, sbuf, obuf):
    wid = lax.axis_index("s") * NC + lax.axis_index("c")
    lane = lax.broadcasted_iota(jnp.int32, (L,), 0)

    @pl.when(wid < B * CB)
    def _():
        b = wid // CB
        cb = wid % CB
        pltpu.sync_copy(part4.at[b, :, :, pl.ds(cb * L, L)], sbuf)
        for v in range(1, NSEG):
            red = sbuf[0, v, :]
            for t in range(1, NW):
                red = jnp.maximum(red, sbuf[t, v, :])
            plsc.store_scatter(obuf, [lane, jnp.full((L,), v - 1, jnp.int32)], red)
        pltpu.sync_copy(obuf, out3.at[b, pl.ds(cb * L, L), :])


def kernel(encoded, masks):
    TH, TW = H // 8, W // 128
    enc = (
        encoded.reshape(B, C, TH, 8, TW, 128)
        .transpose(0, 1, 2, 4, 3, 5)
        .reshape(B, C, HW)
    )
    msk = (
        masks.reshape(B, TH, 8, TW, 128)
        .transpose(0, 1, 3, 2, 4)
        .reshape(B, HW)
    )
    mesh = plsc.VectorSubcoreMesh(core_axis_name="c", subcore_axis_name="s")

    params = pltpu.CompilerParams(
        use_tc_tiling_on_sc=False, needs_layout_passes=False
    )

    seg = pl.kernel(
        _segmax_body,
        out_type=jax.ShapeDtypeStruct((B * NW * PART,), jnp.float32),
        mesh=mesh,
        compiler_params=params,
        scratch_types=[
            pltpu.VMEM((C, P), jnp.float32),
            pltpu.VMEM((C, P), jnp.float32),
            pltpu.VMEM((P,), jnp.int32),
            pltpu.VMEM((P,), jnp.int32),
            pltpu.VMEM((ACCW,), jnp.float32),
            pltpu.VMEM((PART,), jnp.float32),
            pltpu.SemaphoreType.DMA,
            pltpu.SemaphoreType.DMA,
            pltpu.SemaphoreType.DMA,
            pltpu.SemaphoreType.DMA,
        ],
    )
    partials = seg(enc, msk).reshape(B, NW, NSEG, C)

    merge = pl.kernel(
        _merge_body,
        out_type=jax.ShapeDtypeStruct((B, C, V), jnp.float32),
        mesh=mesh,
        compiler_params=params,
        scratch_types=[
            pltpu.VMEM((NW, NSEG, L), jnp.float32),
            pltpu.VMEM((L, V), jnp.float32),
        ],
    )
    out3 = merge(partials)
    return out3[..., None]


# speedup vs baseline: 11.9620x; 1.0037x over previous
import functools

import jax
import jax.numpy as jnp
from jax import lax
from jax.experimental import pallas as pl
from jax.experimental.pallas import tpu as pltpu
from jax.experimental.pallas import tpu_sc as plsc

B, C, H, W = 4, 96, 384, 384
HW = H * W
V = 32
NSEG = V + 1
NC, NS, L = 2, 16, 16
NW = NC * NS
PPT = HW // NW
P = 256
NCHUNK = PPT // P
GRP = P // L
ACCW = C * NSEG * L
PART = NSEG * C
CB = C // L


def _segmax_body(
    enc, msk, part_out,
    dbuf0, dbuf1, mbuf0, mbuf1, acc, pbuf,
    semd0, semd1, semm0, semm1,
):
    wid = lax.axis_index("s") * NC + lax.axis_index("c")
    lane = lax.broadcasted_iota(jnp.int32, (L,), 0)
    ninf = jnp.full((L,), -jnp.inf, dtype=jnp.float32)
    lane528 = lane * (NSEG * L)

    def start(b, k, dref, mref, semd, semm):
        off = wid * PPT + k * P
        pltpu.async_copy(enc.at[b, :, pl.ds(off, P)], dref, semd)
        pltpu.async_copy(msk.at[b, pl.ds(off, P)], mref, semm)

    def wait(dref, mref, semd, semm):
        pltpu.make_async_copy(enc.at[0, :, pl.ds(0, P)], dref, semd).wait()
        pltpu.make_async_copy(msk.at[0, pl.ds(0, P)], mref, semm).wait()

    def process(dref, mref):
        def grp_body(g, carry3):
            m = mref[pl.ds(g * L, L)]
            idx0 = m * L + lane

            @plsc.parallel_loop(0, C, 1, unroll=8)
            def _cbody(c):
                d = dref[c, pl.ds(g * L, L)]
                accv = acc.at[pl.ds(c * (NSEG * L), NSEG * L)]
                old = plsc.load_gather(accv, [idx0])
                plsc.store_scatter(accv, [idx0], jnp.maximum(old, d))

            return carry3

        lax.fori_loop(0, GRP, grp_body, 0)

    def batch_body(b, carry):
        @plsc.parallel_loop(0, ACCW // L, 1, unroll=8)
        def _init(i):
            acc[pl.ds(i * L, L)] = ninf

        start(b, 0, dbuf0, mbuf0, semd0, semm0)

        def pair_body(kk, carry2):
            wait(dbuf0, mbuf0, semd0, semm0)
            start(b, 2 * kk + 1, dbuf1, mbuf1, semd1, semm1)
            process(dbuf0, mbuf0)
            wait(dbuf1, mbuf1, semd1, semm1)

            @pl.when(kk < NCHUNK // 2 - 1)
            def _():
                start(b, 2 * kk + 2, dbuf0, mbuf0, semd0, semm0)

            process(dbuf1, mbuf1)
            return carry2

        lax.fori_loop(0, NCHUNK // 2, pair_body, 0)

        @plsc.parallel_loop(0, NSEG * CB, 1, unroll=2)
        def _flush(t):
            v = t // CB
            cb = t % CB
            sbase = cb * (L * NSEG * L) + v * L
            red = plsc.load_gather(acc, [lane528 + jnp.full((L,), sbase, jnp.int32)])
            for j in range(1, L):
                red = jnp.maximum(
                    red,
                    plsc.load_gather(acc, [lane528 + jnp.full((L,), sbase + j, jnp.int32)]),
                )
            pbuf[pl.ds(v * C + cb * L, L)] = red

        pltpu.sync_copy(pbuf, part_out.at[pl.ds((b * NW + wid) * PART, PART)])
        return carry

    lax.fori_loop(0, B, batch_body, 0)


def _merge_body(part4, out3, sbuf, obuf):
    wid = lax.axis_index("s") * NC + lax.axis_index("c")
    lane = lax.broadcasted_iota(jnp.int32, (L,), 0)

    @pl.when(wid < B * CB)
    def _():
        b = wid // CB
        cb = wid % CB
        pltpu.sync_copy(part4.at[b, :, :, pl.ds(cb * L, L)], sbuf)
        for v in range(1, NSEG):
            red = sbuf[0, v, :]
            for t in range(1, NW):
                red = jnp.maximum(red, sbuf[t, v, :])
            plsc.store_scatter(obuf, [lane, jnp.full((L,), v - 1, jnp.int32)], red)
        pltpu.sync_copy(obuf, out3.at[b, pl.ds(cb * L, L), :])


def kernel(encoded, masks):
    TH, TW = H // 8, W // 128
    enc = (
        encoded.reshape(B, C, TH, 8, TW, 128)
        .transpose(0, 1, 2, 4, 3, 5)
        .reshape(B, C, HW)
    )
    msk = (
        masks.reshape(B, TH, 8, TW, 128)
        .transpose(0, 1, 3, 2, 4)
        .reshape(B, HW)
    )
    mesh = plsc.VectorSubcoreMesh(core_axis_name="c", subcore_axis_name="s")

    params = pltpu.CompilerParams(
        use_tc_tiling_on_sc=False, needs_layout_passes=False
    )

    seg = pl.kernel(
        _segmax_body,
        out_type=jax.ShapeDtypeStruct((B * NW * PART,), jnp.float32),
        mesh=mesh,
        compiler_params=params,
        scratch_types=[
            pltpu.VMEM((C, P), jnp.float32),
            pltpu.VMEM((C, P), jnp.float32),
            pltpu.VMEM((P,), jnp.int32),
            pltpu.VMEM((P,), jnp.int32),
            pltpu.VMEM((ACCW,), jnp.float32),
            pltpu.VMEM((PART,), jnp.float32),
            pltpu.SemaphoreType.DMA,
            pltpu.SemaphoreType.DMA,
            pltpu.SemaphoreType.DMA,
            pltpu.SemaphoreType.DMA,
        ],
    )
    partials = seg(enc, msk).reshape(B, NW, NSEG, C)

    merge = pl.kernel(
        _merge_body,
        out_type=jax.ShapeDtypeStruct((B, C, V), jnp.float32),
        mesh=mesh,
        compiler_params=params,
        scratch_types=[
            pltpu.VMEM((NW, NSEG, L), jnp.float32),
            pltpu.VMEM((L, V), jnp.float32),
        ],
    )
    out3 = merge(partials)
    return out3[..., None]


# speedup vs baseline: 12.2312x; 1.0225x over previous
import functools

import jax
import jax.numpy as jnp
from jax import lax
from jax.experimental import pallas as pl
from jax.experimental.pallas import tpu as pltpu
from jax.experimental.pallas import tpu_sc as plsc

B, C, H, W = 4, 96, 384, 384
HW = H * W
V = 32
NSEG = V + 1
NC, NS, L = 2, 16, 16
NW = NC * NS
PPT = HW // NW
P = 256
NCHUNK = PPT // P
GRP = P // L
ACCW = C * NSEG * L
PART = NSEG * C
CB = C // L


def _segmax_body(
    enc, msk, part_out,
    dbuf0, dbuf1, dbuf2, mbuf0, mbuf1, mbuf2, acc, pbuf,
    semd0, semd1, semd2, semm0, semm1, semm2,
):
    wid = lax.axis_index("s") * NC + lax.axis_index("c")
    lane = lax.broadcasted_iota(jnp.int32, (L,), 0)
    ninf = jnp.full((L,), -jnp.inf, dtype=jnp.float32)
    lane528 = lane * (NSEG * L)

    dbufs = (dbuf0, dbuf1, dbuf2)
    mbufs = (mbuf0, mbuf1, mbuf2)
    semds = (semd0, semd1, semd2)
    semms = (semm0, semm1, semm2)
    TOT = B * NCHUNK

    def start(g, slot):
        b = g // NCHUNK
        off = wid * PPT + (g % NCHUNK) * P
        pltpu.async_copy(enc.at[b, :, pl.ds(off, P)], dbufs[slot], semds[slot])
        pltpu.async_copy(msk.at[b, pl.ds(off, P)], mbufs[slot], semms[slot])

    def wait(slot):
        pltpu.make_async_copy(
            enc.at[0, :, pl.ds(0, P)], dbufs[slot], semds[slot]
        ).wait()
        pltpu.make_async_copy(
            msk.at[0, pl.ds(0, P)], mbufs[slot], semms[slot]
        ).wait()

    def process(dref, mref):
        def grp_body(g, carry3):
            m = mref[pl.ds(g * L, L)]
            idx0 = m * L + lane

            @plsc.parallel_loop(0, C, 1, unroll=8)
            def _cbody(c):
                d = dref[c, pl.ds(g * L, L)]
                accv = acc.at[pl.ds(c * (NSEG * L), NSEG * L)]
                old = plsc.load_gather(accv, [idx0])
                plsc.store_scatter(accv, [idx0], jnp.maximum(old, d))

            return carry3

        lax.fori_loop(0, GRP, grp_body, 0)

    def reset_acc():
        @plsc.parallel_loop(0, ACCW // L, 1, unroll=8)
        def _init(i):
            acc[pl.ds(i * L, L)] = ninf

    def flush(b):
        @plsc.parallel_loop(0, NSEG * CB, 1, unroll=2)
        def _flush(t):
            v = t // CB
            cb = t % CB
            sbase = cb * (L * NSEG * L) + v * L
            red = plsc.load_gather(acc, [lane528 + jnp.full((L,), sbase, jnp.int32)])
            for j in range(1, L):
                red = jnp.maximum(
                    red,
                    plsc.load_gather(acc, [lane528 + jnp.full((L,), sbase + j, jnp.int32)]),
                )
            pbuf[pl.ds(v * C + cb * L, L)] = red

        pltpu.sync_copy(pbuf, part_out.at[pl.ds((b * NW + wid) * PART, PART)])

    start(0, 0)
    start(1, 1)
    reset_acc()

    def chunk_step(g, slot):
        wait(slot)

        @pl.when(g + 2 < TOT)
        def _():
            start(g + 2, (slot + 2) % 3)

        process(dbufs[slot], mbufs[slot])

    def macro_body(j, carry):
        g0 = j * 3
        chunk_step(g0, 0)
        chunk_step(g0 + 1, 1)
        chunk_step(g0 + 2, 2)

        @pl.when(j % (NCHUNK // 3) == NCHUNK // 3 - 1)
        def _():
            flush(j // (NCHUNK // 3))
            reset_acc()

        return carry

    lax.fori_loop(0, B * NCHUNK // 3, macro_body, 0)


def _merge_body(part4, out3, sbuf, obuf):
    wid = lax.axis_index("s") * NC + lax.axis_index("c")
    lane = lax.broadcasted_iota(jnp.int32, (L,), 0)

    @pl.when(wid < B * CB)
    def _():
        b = wid // CB
        cb = wid % CB
        pltpu.sync_copy(part4.at[b, :, :, pl.ds(cb * L, L)], sbuf)
        for v in range(1, NSEG):
            red = sbuf[0, v, :]
            for t in range(1, NW):
                red = jnp.maximum(red, sbuf[t, v, :])
            plsc.store_scatter(obuf, [lane, jnp.full((L,), v - 1, jnp.int32)], red)
        pltpu.sync_copy(obuf, out3.at[b, pl.ds(cb * L, L), :])


def kernel(encoded, masks):
    TH, TW = H // 8, W // 128
    enc = (
        encoded.reshape(B, C, TH, 8, TW, 128)
        .transpose(0, 1, 2, 4, 3, 5)
        .reshape(B, C, HW)
    )
    msk = (
        masks.reshape(B, TH, 8, TW, 128)
        .transpose(0, 1, 3, 2, 4)
        .reshape(B, HW)
    )
    mesh = plsc.VectorSubcoreMesh(core_axis_name="c", subcore_axis_name="s")

    params = pltpu.CompilerParams(
        use_tc_tiling_on_sc=False, needs_layout_passes=False
    )

    seg = pl.kernel(
        _segmax_body,
        out_type=jax.ShapeDtypeStruct((B * NW * PART,), jnp.float32),
        mesh=mesh,
        compiler_params=params,
        scratch_types=[
            pltpu.VMEM((C, P), jnp.float32),
            pltpu.VMEM((C, P), jnp.float32),
            pltpu.VMEM((C, P), jnp.float32),
            pltpu.VMEM((P,), jnp.int32),
            pltpu.VMEM((P,), jnp.int32),
            pltpu.VMEM((P,), jnp.int32),
            pltpu.VMEM((ACCW,), jnp.float32),
            pltpu.VMEM((PART,), jnp.float32),
            pltpu.SemaphoreType.DMA,
            pltpu.SemaphoreType.DMA,
            pltpu.SemaphoreType.DMA,
            pltpu.SemaphoreType.DMA,
            pltpu.SemaphoreType.DMA,
            pltpu.SemaphoreType.DMA,
        ],
    )
    partials = seg(enc, msk).reshape(B, NW, NSEG, C)

    merge = pl.kernel(
        _merge_body,
        out_type=jax.ShapeDtypeStruct((B, C, V), jnp.float32),
        mesh=mesh,
        compiler_params=params,
        scratch_types=[
            pltpu.VMEM((NW, NSEG, L), jnp.float32),
            pltpu.VMEM((L, V), jnp.float32),
        ],
    )
    out3 = merge(partials)
    return out3[..., None]
